# baseline (speedup 1.0000x reference)
import functools

import jax
import jax.numpy as jnp
from jax import lax
from jax.experimental import pallas as pl
from jax.experimental.pallas import tpu as pltpu
from jax.experimental.pallas import tpu_sc as plsc

_N = 10000
_E = 320000
_M = 16
_MID = 32
_TE = 3200
_NH = 2
_EH = _E // _NH
_GCHUNK = 5000


def _gather_sc(table, idx, n_edges):
    info = plsc.get_sparse_core_info()
    nw = info.num_cores * info.num_subcores
    b_per_w = n_edges // nw
    n_chunks = b_per_w // _GCHUNK
    mesh = plsc.VectorSubcoreMesh(core_axis_name="c", subcore_axis_name="s")

    @functools.partial(
        pl.kernel,
        mesh=mesh,
        out_type=jax.ShapeDtypeStruct((n_edges, _M), jnp.float32),
        scratch_types=[
            pltpu.VMEM((_GCHUNK,), jnp.int32),
            pltpu.VMEM((_GCHUNK, _M), jnp.float32),
            pltpu.SemaphoreType.DMA,
        ],
        compiler_params=pltpu.CompilerParams(use_tc_tiling_on_sc=False),
    )
    def gather_kernel(table_hbm, idx_hbm, out_hbm, idx_v, rows_v, sem):
        wid = lax.axis_index("s") * info.num_cores + lax.axis_index("c")
        base = wid * b_per_w
        for c in range(n_chunks):
            off = base + c * _GCHUNK
            pltpu.sync_copy(idx_hbm.at[pl.ds(off, _GCHUNK)], idx_v)
            pltpu.async_copy(table_hbm.at[idx_v], rows_v, sem).wait()
            pltpu.sync_copy(rows_v, out_hbm.at[pl.ds(off, _GCHUNK)])

    return gather_kernel(table, idx)


def _tc_body(ea_ref, r_ref, bs_ref, g_ref, w1a_ref, w1rb_ref, jm_ref,
             gb1_ref, w2b_ref, gb2_ref, w3_ref, b3_ref, tm_ref, sm_ref,
             out_ref):
    f32 = jnp.float32
    bf = jnp.bfloat16
    jm = jm_ref[...]
    ones_row = jnp.ones((1, _TE), dtype=f32)

    def dotf(a, b):
        return jnp.dot(a, b, preferred_element_type=f32)

    def ln_relu(y, gb):
        mu = dotf(jm, y)
        s2 = dotf(jm, y * y)
        ga = dotf(gb[0:1, :].T, ones_row)
        be = dotf(gb[1:2, :].T, ones_row)
        return jnp.maximum((y - mu) * lax.rsqrt(s2 - mu * mu + 1e-5)
                           * ga + be, 0.0)

    aug = jnp.concatenate([r_ref[...], ones_row], axis=0)
    y = dotf(w1a_ref[...], ea_ref[...]) + dotf(w1rb_ref[...], aug)
    y = ln_relu(y, gb1_ref[...])
    y = dotf(w2b_ref[...][:, :_MID], y) \
        + dotf(w2b_ref[...][:, _MID:], ones_row)
    y = ln_relu(y, gb2_ref[...])
    y3 = dotf(w3_ref[...], y.astype(bf)) + dotf(b3_ref[...], ones_row)
    grep = dotf(tm_ref[...], g_ref[...].astype(bf))
    out = dotf(sm_ref[...], (y3 * grep).astype(bf))
    out_ref[...] = out * dotf(jnp.ones((_M, 1), dtype=f32), bs_ref[...])


def kernel(h0, r, edge_attr, basis_00, W1, b1, g1, be1, W2, b2, g2, be2,
           W3, b3, edge_index):
    f32 = jnp.float32
    table = h0.reshape(_N, _M)
    src_idx = edge_index[0]

    eaT = edge_attr.T
    rT = r.T
    bsT = basis_00.reshape(1, _E)

    eye16 = jnp.eye(_M, dtype=f32)
    consts = [
        W1[:, :16],
        jnp.stack([W1[:, 16], b1], axis=1),
        jnp.full((_MID, _MID), 1.0 / _MID, dtype=f32),
        jnp.stack([g1, be1], axis=0),
        jnp.concatenate([W2, b2[:, None]], axis=1),
        jnp.stack([g2, be2], axis=0),
        W3.astype(jnp.bfloat16),
        b3.reshape(256, 1),
        jnp.tile(eye16, (_M, 1)).astype(jnp.bfloat16),
        jnp.kron(eye16, jnp.ones((1, _M))).astype(jnp.bfloat16),
    ]

    full_spec = lambda a: pl.BlockSpec(a.shape, lambda i: (0,) * a.ndim)
    nblk = _EH // _TE

    halves = []
    for h in range(_NH):
        idx_h = lax.slice_in_dim(src_idx, h * _EH, (h + 1) * _EH)
        gT_h = _gather_sc(table, idx_h, _EH).T
        off_spec = lambda w, hh=h: pl.BlockSpec(
            (w, _TE), lambda i, _hh=hh: (0, i + _hh * nblk))
        loc_spec = lambda w: pl.BlockSpec((w, _TE), lambda i: (0, i))
        outT_h = pl.pallas_call(
            _tc_body,
            grid=(nblk,),
            in_specs=[off_spec(_M), off_spec(1), off_spec(1), loc_spec(_M)]
                     + [full_spec(a) for a in consts],
            out_specs=loc_spec(_M),
            out_shape=jax.ShapeDtypeStruct((_M, _EH), f32),
            compiler_params=pltpu.CompilerParams(
                dimension_semantics=("arbitrary",)),
        )(eaT, rT, bsT, gT_h, *consts)
        halves.append(outT_h)

    outT = jnp.concatenate(halves, axis=1)
    return outT.T.reshape(_E, _M, 1)


# speedup vs baseline: 1.0908x; 1.0908x over previous
import functools

import jax
import jax.numpy as jnp
from jax import lax
from jax.experimental import pallas as pl
from jax.experimental.pallas import tpu as pltpu
from jax.experimental.pallas import tpu_sc as plsc

_N = 10000
_E = 320000
_M = 16
_MID = 32
_TE = 6400
_NH = 2
_EH = _E // _NH
_GCHUNK = 5000


def _gather_sc(table, idx, n_edges):
    info = plsc.get_sparse_core_info()
    nw = info.num_cores * info.num_subcores
    b_per_w = n_edges // nw
    n_chunks = b_per_w // _GCHUNK
    mesh = plsc.VectorSubcoreMesh(core_axis_name="c", subcore_axis_name="s")

    @functools.partial(
        pl.kernel,
        mesh=mesh,
        out_type=jax.ShapeDtypeStruct((n_edges, _M), jnp.float32),
        scratch_types=[
            pltpu.VMEM((_GCHUNK,), jnp.int32),
            pltpu.VMEM((_GCHUNK, _M), jnp.float32),
            pltpu.SemaphoreType.DMA,
        ],
        compiler_params=pltpu.CompilerParams(use_tc_tiling_on_sc=False),
    )
    def gather_kernel(table_hbm, idx_hbm, out_hbm, idx_v, rows_v, sem):
        wid = lax.axis_index("s") * info.num_cores + lax.axis_index("c")
        base = wid * b_per_w
        for c in range(n_chunks):
            off = base + c * _GCHUNK
            pltpu.sync_copy(idx_hbm.at[pl.ds(off, _GCHUNK)], idx_v)
            pltpu.async_copy(table_hbm.at[idx_v], rows_v, sem).wait()
            pltpu.sync_copy(rows_v, out_hbm.at[pl.ds(off, _GCHUNK)])

    return gather_kernel(table, idx)


def _tc_body(ea_ref, r_ref, bs_ref, g_ref, w1a_ref, w1rb_ref, jm_ref,
             gb1_ref, w2b_ref, gb2_ref, w3_ref, b3_ref, tm_ref, sm_ref,
             out_ref):
    f32 = jnp.float32
    bf = jnp.bfloat16
    jm = jm_ref[...]
    ones_row = jnp.ones((1, _TE), dtype=f32)

    def dotf(a, b):
        return jnp.dot(a, b, preferred_element_type=f32)

    def ln_relu(y, gb):
        mu = dotf(jm, y)
        s2 = dotf(jm, y * y)
        ga = dotf(gb[0:1, :].T, ones_row)
        be = dotf(gb[1:2, :].T, ones_row)
        return jnp.maximum((y - mu) * lax.rsqrt(s2 - mu * mu + 1e-5)
                           * ga + be, 0.0)

    aug = jnp.concatenate([r_ref[...], ones_row], axis=0)
    y = dotf(w1a_ref[...], ea_ref[...]) + dotf(w1rb_ref[...], aug)
    y = ln_relu(y, gb1_ref[...])
    y = dotf(w2b_ref[...][:, :_MID], y) \
        + dotf(w2b_ref[...][:, _MID:], ones_row)
    y = ln_relu(y, gb2_ref[...])
    y3 = dotf(w3_ref[...], y.astype(bf)) + dotf(b3_ref[...], ones_row)
    grep = dotf(tm_ref[...], g_ref[...].astype(bf))
    out = dotf(sm_ref[...], (y3 * grep).astype(bf))
    out_ref[...] = out * dotf(jnp.ones((_M, 1), dtype=f32), bs_ref[...])


def kernel(h0, r, edge_attr, basis_00, W1, b1, g1, be1, W2, b2, g2, be2,
           W3, b3, edge_index):
    f32 = jnp.float32
    table = h0.reshape(_N, _M)
    src_idx = edge_index[0]

    eaT = edge_attr.T
    rT = r.T
    bsT = basis_00.reshape(1, _E)

    eye16 = jnp.eye(_M, dtype=f32)
    consts = [
        W1[:, :16],
        jnp.stack([W1[:, 16], b1], axis=1),
        jnp.full((_MID, _MID), 1.0 / _MID, dtype=f32),
        jnp.stack([g1, be1], axis=0),
        jnp.concatenate([W2, b2[:, None]], axis=1),
        jnp.stack([g2, be2], axis=0),
        W3.astype(jnp.bfloat16),
        b3.reshape(256, 1),
        jnp.tile(eye16, (_M, 1)).astype(jnp.bfloat16),
        jnp.kron(eye16, jnp.ones((1, _M))).astype(jnp.bfloat16),
    ]

    full_spec = lambda a: pl.BlockSpec(a.shape, lambda i: (0,) * a.ndim)
    nblk = _EH // _TE

    halves = []
    for h in range(_NH):
        idx_h = lax.slice_in_dim(src_idx, h * _EH, (h + 1) * _EH)
        gT_h = _gather_sc(table, idx_h, _EH).T
        off_spec = lambda w, hh=h: pl.BlockSpec(
            (w, _TE), lambda i, _hh=hh: (0, i + _hh * nblk))
        loc_spec = lambda w: pl.BlockSpec((w, _TE), lambda i: (0, i))
        outT_h = pl.pallas_call(
            _tc_body,
            grid=(nblk,),
            in_specs=[off_spec(_M), off_spec(1), off_spec(1), loc_spec(_M)]
                     + [full_spec(a) for a in consts],
            out_specs=loc_spec(_M),
            out_shape=jax.ShapeDtypeStruct((_M, _EH), f32),
            compiler_params=pltpu.CompilerParams(
                dimension_semantics=("arbitrary",)),
        )(eaT, rT, bsT, gT_h, *consts)
        halves.append(outT_h)

    outT = jnp.concatenate(halves, axis=1)
    return outT.T.reshape(_E, _M, 1)
